# Initial kernel scaffold; baseline (speedup 1.0000x reference)
import jax
import jax.numpy as jnp
from jax.experimental import pallas as pl


def kernel(X, A, W, b):
    raise NotImplementedError("write your pallas kernel here")


# speedup vs baseline: 11.9333x; 11.9333x over previous
import functools

import jax
import jax.numpy as jnp
from jax import lax
from jax.experimental import pallas as pl
from jax.experimental.pallas import tpu as pltpu
from jax.experimental.pallas import tpu_sc as plsc

N = 10000
D = 128
E = 320000

NC = 2
NS = 16
NW = NC * NS
CHUNK = 128
N_CHUNKS = 80
EPW = N_CHUNKS * CHUNK
E_PAD = NW * EPW
N_PAD = 10112
ROWS_PW = N_PAD // NS

_mesh = plsc.VectorSubcoreMesh(
    core_axis_name="c", subcore_axis_name="s", num_cores=NC, num_subcores=NS
)


@functools.partial(
    pl.kernel,
    out_type=jax.ShapeDtypeStruct((NC, N_PAD, D), jnp.float32),
    mesh=_mesh,
    scratch_types=[
        pltpu.VMEM((N_CHUNKS, CHUNK), jnp.int32),
        pltpu.VMEM((CHUNK, D), jnp.float32),
        pltpu.VMEM_SHARED((N_PAD, D), jnp.float32),
    ],
)
def _deg_kernel(dst_hbm, ones_hbm, zeros_hbm, out_hbm, idx_v, ones_v, deg_sh):
    c = lax.axis_index("c")
    s = lax.axis_index("s")
    wid = s * NC + c

    pltpu.sync_copy(ones_hbm, ones_v)
    pltpu.sync_copy(
        zeros_hbm.at[pl.ds(s * ROWS_PW, ROWS_PW)],
        deg_sh.at[pl.ds(s * ROWS_PW, ROWS_PW)],
    )
    pltpu.sync_copy(dst_hbm.at[wid], idx_v)
    plsc.subcore_barrier()

    def _body(j, carry):
        pltpu.sync_copy(ones_v, deg_sh.at[idx_v.at[j]], add=True)
        return carry

    lax.fori_loop(0, N_CHUNKS, _body, 0)
    plsc.subcore_barrier()

    pltpu.sync_copy(
        deg_sh.at[pl.ds(s * ROWS_PW, ROWS_PW)],
        out_hbm.at[c, pl.ds(s * ROWS_PW, ROWS_PW)],
    )


@functools.partial(
    pl.kernel,
    out_type=jax.ShapeDtypeStruct((NC, N_PAD, D), jnp.float32),
    mesh=_mesh,
    scratch_types=[
        pltpu.VMEM((N_CHUNKS, CHUNK), jnp.int32),
        pltpu.VMEM((N_CHUNKS, CHUNK), jnp.int32),
        pltpu.VMEM((CHUNK, D), jnp.float32),
        pltpu.VMEM_SHARED((N_PAD, D), jnp.float32),
        pltpu.SemaphoreType.DMA,
    ],
)
def _agg_kernel(y_hbm, src_hbm, dst_hbm, zeros_hbm, out_hbm,
                srcv, dstv, rows, agg_sh, sem):
    c = lax.axis_index("c")
    s = lax.axis_index("s")
    wid = s * NC + c

    pltpu.sync_copy(
        zeros_hbm.at[pl.ds(s * ROWS_PW, ROWS_PW)],
        agg_sh.at[pl.ds(s * ROWS_PW, ROWS_PW)],
    )
    pltpu.sync_copy(src_hbm.at[wid], srcv)
    pltpu.sync_copy(dst_hbm.at[wid], dstv)
    plsc.subcore_barrier()

    def _body(j, carry):
        pltpu.async_copy(y_hbm.at[srcv.at[j]], rows, sem).wait()
        pltpu.sync_copy(rows, agg_sh.at[dstv.at[j]], add=True)
        return carry

    lax.fori_loop(0, N_CHUNKS, _body, 0)
    plsc.subcore_barrier()

    pltpu.sync_copy(
        agg_sh.at[pl.ds(s * ROWS_PW, ROWS_PW)],
        out_hbm.at[c, pl.ds(s * ROWS_PW, ROWS_PW)],
    )


_R = 1000


def _prep_body(degp_ref, x_ref, y_ref):
    deg = degp_ref[0, :, 0] + degp_ref[1, :, 0] + 1.0
    dis = lax.rsqrt(deg)
    y_ref[...] = x_ref[...] * dis[:, None]


def _out_body(degp_ref, y_ref, p_ref, w_ref, b_ref, o_ref):
    deg = degp_ref[0, :, 0] + degp_ref[1, :, 0] + 1.0
    dis = lax.rsqrt(deg)
    t = y_ref[...] + p_ref[0] + p_ref[1]
    acc = jnp.dot(t, w_ref[...], preferred_element_type=jnp.float32)
    acc = acc * dis[:, None] + b_ref[...]
    o_ref[...] = jnp.where(acc > 0, acc, 0.01 * acc)


def kernel(X, A, W, b):
    src = A[0].astype(jnp.int32)
    dst = A[1].astype(jnp.int32)
    pad = jnp.full((E_PAD - E,), N, dtype=jnp.int32)
    src_p = jnp.concatenate([src, pad]).reshape(NW, N_CHUNKS, CHUNK)
    dst_p = jnp.concatenate([dst, pad]).reshape(NW, N_CHUNKS, CHUNK)

    onesD = jnp.ones((CHUNK, D), jnp.float32)
    zerosD = jnp.zeros((N_PAD, D), jnp.float32)

    degp = _deg_kernel(dst_p, onesD, zerosD)

    y = pl.pallas_call(
        _prep_body,
        grid=(N // _R,),
        in_specs=[
            pl.BlockSpec((2, _R, D), lambda i: (0, i, 0)),
            pl.BlockSpec((_R, D), lambda i: (i, 0)),
        ],
        out_specs=pl.BlockSpec((_R, D), lambda i: (i, 0)),
        out_shape=jax.ShapeDtypeStruct((N_PAD, D), jnp.float32),
    )(degp, X)

    partials = _agg_kernel(y, src_p, dst_p, zerosD)

    out = pl.pallas_call(
        _out_body,
        grid=(N // _R,),
        in_specs=[
            pl.BlockSpec((2, _R, D), lambda i: (0, i, 0)),
            pl.BlockSpec((_R, D), lambda i: (i, 0)),
            pl.BlockSpec((2, _R, D), lambda i: (0, i, 0)),
            pl.BlockSpec((D, D), lambda i: (0, 0)),
            pl.BlockSpec((1, D), lambda i: (0, 0)),
        ],
        out_specs=pl.BlockSpec((_R, D), lambda i: (i, 0)),
        out_shape=jax.ShapeDtypeStruct((N, D), jnp.float32),
    )(degp, y, partials, W, b.reshape(1, D))

    return out


# speedup vs baseline: 13.2038x; 1.1065x over previous
import functools

import jax
import jax.numpy as jnp
from jax import lax
from jax.experimental import pallas as pl
from jax.experimental.pallas import tpu as pltpu
from jax.experimental.pallas import tpu_sc as plsc

N = 10000
D = 128
E = 320000

NC = 2
NS = 16
NW = NC * NS
CHUNK = 128
N_CHUNKS = 80
EPW = N_CHUNKS * CHUNK
E_PAD = NW * EPW
N_PAD = 10112
ROWS_PW = N_PAD // NS

_mesh = plsc.VectorSubcoreMesh(
    core_axis_name="c", subcore_axis_name="s", num_cores=NC, num_subcores=NS
)


@functools.partial(
    pl.kernel,
    out_type=jax.ShapeDtypeStruct((NC, N_PAD, D), jnp.float32),
    mesh=_mesh,
    scratch_types=[
        pltpu.VMEM((N_CHUNKS, CHUNK), jnp.int32),
        pltpu.VMEM((CHUNK, D), jnp.float32),
        pltpu.VMEM_SHARED((N_PAD, D), jnp.float32),
    ],
)
def _deg_kernel(dst_hbm, ones_hbm, zeros_hbm, out_hbm, idx_v, ones_v, deg_sh):
    c = lax.axis_index("c")
    s = lax.axis_index("s")
    wid = s * NC + c

    pltpu.sync_copy(ones_hbm, ones_v)
    pltpu.sync_copy(
        zeros_hbm.at[pl.ds(s * ROWS_PW, ROWS_PW)],
        deg_sh.at[pl.ds(s * ROWS_PW, ROWS_PW)],
    )
    pltpu.sync_copy(dst_hbm.at[wid], idx_v)
    plsc.subcore_barrier()

    def _body(j, carry):
        pltpu.sync_copy(ones_v, deg_sh.at[idx_v.at[j]], add=True)
        return carry

    lax.fori_loop(0, N_CHUNKS, _body, 0)
    plsc.subcore_barrier()

    pltpu.sync_copy(
        deg_sh.at[pl.ds(s * ROWS_PW, ROWS_PW)],
        out_hbm.at[c, pl.ds(s * ROWS_PW, ROWS_PW)],
    )


HALF = N_CHUNKS // 2


@functools.partial(
    pl.kernel,
    out_type=jax.ShapeDtypeStruct((NC, N_PAD, D), jnp.float32),
    mesh=_mesh,
    scratch_types=[
        pltpu.VMEM((HALF, CHUNK), jnp.int32),
        pltpu.VMEM((HALF, CHUNK), jnp.int32),
        pltpu.VMEM((CHUNK, D), jnp.float32),
        pltpu.VMEM((CHUNK, D), jnp.float32),
        pltpu.VMEM_SHARED((N_PAD, D), jnp.float32),
        pltpu.SemaphoreType.DMA,
        pltpu.SemaphoreType.DMA,
    ],
)
def _agg_kernel(y_hbm, src_hbm, dst_hbm, zeros_hbm, out_hbm,
                srcv, dstv, rows0, rows1, agg_sh, gsem, ssem):
    c = lax.axis_index("c")
    s = lax.axis_index("s")
    wid = s * NC + c
    rows = (rows0, rows1)

    pltpu.sync_copy(
        zeros_hbm.at[pl.ds(s * ROWS_PW, ROWS_PW)],
        agg_sh.at[pl.ds(s * ROWS_PW, ROWS_PW)],
    )
    plsc.subcore_barrier()

    for h in range(2):
        pltpu.sync_copy(src_hbm.at[wid, pl.ds(h * HALF, HALF)], srcv)
        pltpu.sync_copy(dst_hbm.at[wid, pl.ds(h * HALF, HALF)], dstv)
        pltpu.async_copy(y_hbm.at[srcv.at[0]], rows[0], gsem)

        def _group(g, carry):
            for b in range(2):
                j = g * 2 + b
                bn = 1 - b
                pltpu.make_async_copy(
                    y_hbm.at[srcv.at[j]], rows[b], gsem
                ).wait()

                @pl.when(j > 0)
                def _():
                    pltpu.make_async_copy(
                        rows[bn], agg_sh.at[dstv.at[j - 1]], ssem
                    ).wait()

                @pl.when(j + 1 < HALF)
                def _():
                    pltpu.async_copy(y_hbm.at[srcv.at[j + 1]], rows[bn], gsem)

                pltpu.async_copy(
                    rows[b], agg_sh.at[dstv.at[j]], ssem, add=True
                )
            return carry

        lax.fori_loop(0, HALF // 2, _group, 0)
        pltpu.make_async_copy(
            rows[(HALF - 1) % 2], agg_sh.at[dstv.at[HALF - 1]], ssem
        ).wait()
    plsc.subcore_barrier()

    pltpu.sync_copy(
        agg_sh.at[pl.ds(s * ROWS_PW, ROWS_PW)],
        out_hbm.at[c, pl.ds(s * ROWS_PW, ROWS_PW)],
    )


_R = 1000


def _prep_body(dga_ref, dgb_ref, x_ref, y_ref):
    deg = dga_ref[...] + dgb_ref[...] + 1.0
    dis = lax.rsqrt(deg)
    y_ref[...] = x_ref[...] * dis


def _out_body(dga_ref, dgb_ref, y_ref, p_ref, w_ref, b_ref, o_ref):
    deg = dga_ref[...] + dgb_ref[...] + 1.0
    dis = lax.rsqrt(deg)
    t = y_ref[...] + p_ref[0] + p_ref[1]
    acc = jnp.dot(t, w_ref[...], preferred_element_type=jnp.float32)
    acc = acc * dis + b_ref[...]
    o_ref[...] = jnp.where(acc > 0, acc, 0.01 * acc)


def kernel(X, A, W, b):
    src = A[0].astype(jnp.int32)
    dst = A[1].astype(jnp.int32)
    pad = jnp.full((E_PAD - E,), N, dtype=jnp.int32)
    src_p = jnp.concatenate([src, pad]).reshape(NW, N_CHUNKS, CHUNK)
    dst_p = jnp.concatenate([dst, pad]).reshape(NW, N_CHUNKS, CHUNK)

    zerosD = jnp.zeros((N_PAD, D), jnp.float32)
    onesD = jnp.ones((CHUNK, D), jnp.float32)

    degp = _deg_kernel(dst_p, onesD, zerosD)
    dga = degp[0, :, :1]
    dgb = degp[1, :, :1]

    y = pl.pallas_call(
        _prep_body,
        grid=(N // _R,),
        in_specs=[
            pl.BlockSpec((_R, 1), lambda i: (i, 0)),
            pl.BlockSpec((_R, 1), lambda i: (i, 0)),
            pl.BlockSpec((_R, D), lambda i: (i, 0)),
        ],
        out_specs=pl.BlockSpec((_R, D), lambda i: (i, 0)),
        out_shape=jax.ShapeDtypeStruct((N_PAD, D), jnp.float32),
    )(dga, dgb, X)

    partials = _agg_kernel(y, src_p, dst_p, zerosD)

    out = pl.pallas_call(
        _out_body,
        grid=(N // _R,),
        in_specs=[
            pl.BlockSpec((_R, 1), lambda i: (i, 0)),
            pl.BlockSpec((_R, 1), lambda i: (i, 0)),
            pl.BlockSpec((_R, D), lambda i: (i, 0)),
            pl.BlockSpec((2, _R, D), lambda i: (0, i, 0)),
            pl.BlockSpec((D, D), lambda i: (0, 0)),
            pl.BlockSpec((1, D), lambda i: (0, 0)),
        ],
        out_specs=pl.BlockSpec((_R, D), lambda i: (i, 0)),
        out_shape=jax.ShapeDtypeStruct((N, D), jnp.float32),
    )(dga, dgb, y, partials, W, b.reshape(1, D))

    return out
